# baseline (speedup 1.0000x reference)
import jax
import jax.numpy as jnp
from jax.experimental import pallas as pl

_R = 196
_NB = 16
_K = 9
_C = 96


def _block_kernel(x_ref, wl_ref, wr_ref, b_ref, out_ref):
    for i in range(_NB):
        _one_block(i, x_ref, wl_ref, wr_ref, b_ref, out_ref)


def _one_block(i, x_ref, wl_ref, wr_ref, b_ref, out_ref):
    Xc = x_ref[i].reshape(_C, _R)

    x2 = jnp.sum(Xc * Xc, axis=0, keepdims=True)
    h = (0.5 * x2).reshape(_R, 1)
    G = jax.lax.dot_general(
        Xc, Xc, (((0,), (0,)), ((), ())), preferred_element_type=jnp.float32
    )
    E = h - G

    if i == _NB - 1:
        row = jax.lax.broadcasted_iota(jnp.int32, (_R, _R), 0)
        col = jax.lax.broadcasted_iota(jnp.int32, (_R, _R), 1)
        cross = (row == _R - 1) != (col == _R - 1)
        E = jnp.where(cross, jnp.inf, E)

    S = None
    for k in range(_K):
        m = jnp.min(E, axis=0, keepdims=True)
        hit = E == m
        hf = hit.astype(jnp.float32)
        S = hf if S is None else S + hf
        if k != _K - 1:
            E = jnp.where(hit, jnp.inf, E)

    meanT = jax.lax.dot_general(
        Xc, S, (((1,), (0,)), ((), ())),
        preferred_element_type=jnp.float32,
    ) * (1.0 / float(_K))

    if i == _NB - 1:
        hsum = jnp.sum(x_ref[0].reshape(_C, _R)[:, :8], axis=1, keepdims=True)
        fixed = (Xc[:, _R - 1 :] + hsum) * (1.0 / float(_K))
        colv = jax.lax.broadcasted_iota(jnp.int32, (_C, _R), 1)
        meanT = jnp.where(colv == _R - 1, fixed, meanT)

    outT = jax.lax.dot_general(
        wl_ref[...], meanT, (((1,), (0,)), ((), ())),
        preferred_element_type=jnp.float32,
    ) + jax.lax.dot_general(
        wr_ref[...], Xc, (((1,), (0,)), ((), ())),
        preferred_element_type=jnp.float32,
    )
    out_ref[i] = jnp.maximum(outT.T + b_ref[...], 0.0)


def kernel(x, W_l, W_r, b):
    Bs, Cs, Hs, Ws = x.shape
    N = Bs * Hs * Ws
    b2 = b.reshape(1, Cs)

    out = pl.pallas_call(
        _block_kernel,
        grid=(1,),
        in_specs=[
            pl.BlockSpec((_NB, Cs, Hs, Ws), lambda i: (0, 0, 0, 0)),
            pl.BlockSpec((Cs, Cs), lambda i: (0, 0)),
            pl.BlockSpec((Cs, Cs), lambda i: (0, 0)),
            pl.BlockSpec((1, Cs), lambda i: (0, 0)),
        ],
        out_specs=pl.BlockSpec((_NB, _R, Cs), lambda i: (0, 0, 0)),
        out_shape=jax.ShapeDtypeStruct((_NB, _R, Cs), jnp.float32),
    )(x, W_l, W_r, b2)
    return out.reshape(N, Cs)


# speedup vs baseline: 1.0173x; 1.0173x over previous
import jax
import jax.numpy as jnp
from jax.experimental import pallas as pl

_R = 196
_NB = 16
_K = 9
_C = 96


_BPG = 4


def _block_kernel(x_ref, head_ref, wl_ref, wr_ref, b_ref, out_ref):
    for sb in range(_BPG):
        _one_block(_BPG * pl.program_id(0) + sb, sb, x_ref, head_ref,
                   wl_ref, wr_ref, b_ref, out_ref)


def _one_block(i, sb, x_ref, head_ref, wl_ref, wr_ref, b_ref, out_ref):
    Xc = x_ref[sb].reshape(_C, _R)

    x2 = jnp.sum(Xc * Xc, axis=0, keepdims=True)
    h = (0.5 * x2).reshape(_R, 1)
    G = jax.lax.dot_general(
        Xc, Xc, (((0,), (0,)), ((), ())), preferred_element_type=jnp.float32
    )
    E = h - G

    if sb == _BPG - 1:
        is_last = i == _NB - 1
        row = jax.lax.broadcasted_iota(jnp.int32, (_R, _R), 0)
        col = jax.lax.broadcasted_iota(jnp.int32, (_R, _R), 1)
        cross = (row == _R - 1) != (col == _R - 1)
        E = jnp.where(is_last & cross, jnp.inf, E)

    S = None
    for k in range(_K):
        m = jnp.min(E, axis=0, keepdims=True)
        hit = E == m
        hf = hit.astype(jnp.float32)
        S = hf if S is None else S + hf
        if k != _K - 1:
            E = jnp.where(hit, jnp.inf, E)

    meanT = jax.lax.dot_general(
        Xc, S, (((1,), (0,)), ((), ())),
        preferred_element_type=jnp.float32,
    ) * (1.0 / float(_K))

    if sb == _BPG - 1:
        is_last = i == _NB - 1
        hsum = jnp.sum(head_ref[...], axis=1, keepdims=True)
        fixed = (Xc[:, _R - 1 :] + hsum) * (1.0 / float(_K))
        colv = jax.lax.broadcasted_iota(jnp.int32, (_C, _R), 1)
        meanT = jnp.where(is_last & (colv == _R - 1), fixed, meanT)

    outT = jax.lax.dot_general(
        wl_ref[...], meanT, (((1,), (0,)), ((), ())),
        preferred_element_type=jnp.float32,
    ) + jax.lax.dot_general(
        wr_ref[...], Xc, (((1,), (0,)), ((), ())),
        preferred_element_type=jnp.float32,
    )
    out_ref[sb] = jnp.maximum(outT.T + b_ref[...], 0.0)


def kernel(x, W_l, W_r, b):
    Bs, Cs, Hs, Ws = x.shape
    N = Bs * Hs * Ws
    b2 = b.reshape(1, Cs)
    head = x[0, :, 0, :8]

    out = pl.pallas_call(
        _block_kernel,
        grid=(_NB // _BPG,),
        in_specs=[
            pl.BlockSpec((_BPG, Cs, Hs, Ws), lambda i: (i, 0, 0, 0)),
            pl.BlockSpec((Cs, 8), lambda i: (0, 0)),
            pl.BlockSpec((Cs, Cs), lambda i: (0, 0)),
            pl.BlockSpec((Cs, Cs), lambda i: (0, 0)),
            pl.BlockSpec((1, Cs), lambda i: (0, 0)),
        ],
        out_specs=pl.BlockSpec((_BPG, _R, Cs), lambda i: (i, 0, 0)),
        out_shape=jax.ShapeDtypeStruct((_NB, _R, Cs), jnp.float32),
    )(x, head, W_l, W_r, b2)
    return out.reshape(N, Cs)


# speedup vs baseline: 1.5273x; 1.5012x over previous
import jax
import jax.numpy as jnp
from jax.experimental import pallas as pl

_R = 196
_NB = 16
_K = 9
_C = 96


def _block_kernel(x_ref, wl_ref, wr_ref, b_ref, out_ref):
    for i in range(_NB):
        _one_block(i, x_ref, wl_ref, wr_ref, b_ref, out_ref)


def _one_block(i, x_ref, wl_ref, wr_ref, b_ref, out_ref):
    Xc = x_ref[i]

    x2 = jnp.sum(Xc * Xc, axis=0, keepdims=True)
    h = (0.5 * x2).reshape(_R, 1)
    G = jax.lax.dot_general(
        Xc, Xc, (((0,), (0,)), ((), ())), preferred_element_type=jnp.float32
    )
    E = h - G

    if i == _NB - 1:
        row = jax.lax.broadcasted_iota(jnp.int32, (_R, _R), 0)
        col = jax.lax.broadcasted_iota(jnp.int32, (_R, _R), 1)
        cross = (row == _R - 1) != (col == _R - 1)
        E = jnp.where(cross, jnp.inf, E)

    S = None
    for k in range(_K):
        m = jnp.min(E, axis=0, keepdims=True)
        hit = E == m
        hf = hit.astype(jnp.float32)
        S = hf if S is None else S + hf
        if k != _K - 1:
            E = jnp.where(hit, jnp.inf, E)

    meanT = jax.lax.dot_general(
        Xc, S, (((1,), (0,)), ((), ())),
        preferred_element_type=jnp.float32,
    ) * (1.0 / float(_K))

    if i == _NB - 1:
        hsum = jnp.sum(x_ref[0][:, :8], axis=1, keepdims=True)
        fixed = (Xc[:, _R - 1 :] + hsum) * (1.0 / float(_K))
        colv = jax.lax.broadcasted_iota(jnp.int32, (_C, _R), 1)
        meanT = jnp.where(colv == _R - 1, fixed, meanT)

    outT = jax.lax.dot_general(
        wl_ref[...], meanT, (((1,), (0,)), ((), ())),
        preferred_element_type=jnp.float32,
    ) + jax.lax.dot_general(
        wr_ref[...], Xc, (((1,), (0,)), ((), ())),
        preferred_element_type=jnp.float32,
    )
    out_ref[i] = jnp.maximum(outT.T + b_ref[...], 0.0)


def kernel(x, W_l, W_r, b):
    Bs, Cs, Hs, Ws = x.shape
    N = Bs * Hs * Ws
    xc = x.reshape(Bs, Cs, Hs * Ws)
    b2 = b.reshape(1, Cs)

    out = pl.pallas_call(
        _block_kernel,
        grid=(1,),
        in_specs=[
            pl.BlockSpec((_NB, Cs, _R), lambda i: (0, 0, 0)),
            pl.BlockSpec((Cs, Cs), lambda i: (0, 0)),
            pl.BlockSpec((Cs, Cs), lambda i: (0, 0)),
            pl.BlockSpec((1, Cs), lambda i: (0, 0)),
        ],
        out_specs=pl.BlockSpec((_NB, _R, Cs), lambda i: (0, 0, 0)),
        out_shape=jax.ShapeDtypeStruct((_NB, _R, Cs), jnp.float32),
    )(xc, W_l, W_r, b2)
    return out.reshape(N, Cs)
